# baseline (speedup 1.0000x reference)
import functools

import jax
import jax.numpy as jnp
from jax import lax
from jax.experimental import pallas as pl
from jax.experimental.pallas import tpu as pltpu
from jax.experimental.pallas import tpu_sc as plsc


def _emb_lookup(idx, table, *, num_cores, num_subcores, chunk):
    B, = idx.shape
    V, D = table.shape
    nw = num_cores * num_subcores
    b_per_w = B // nw
    nchunks = b_per_w // chunk

    mesh = plsc.VectorSubcoreMesh(core_axis_name="c", subcore_axis_name="s")

    @functools.partial(
        pl.kernel,
        out_type=jax.ShapeDtypeStruct((B, D), jnp.float32),
        mesh=mesh,
        compiler_params=pltpu.CompilerParams(use_tc_tiling_on_sc=False),
        scratch_types=[
            pltpu.VMEM((b_per_w,), jnp.int32),
            pltpu.VMEM((chunk, D), jnp.float32),
            pltpu.SemaphoreType.DMA,
        ],
    )
    def emb(idx_hbm, table_hbm, out_hbm, idx_v, rows, gsem):
        wid = lax.axis_index("s") * num_cores + lax.axis_index("c")
        base = wid * b_per_w
        pltpu.sync_copy(idx_hbm.at[pl.ds(base, b_per_w)], idx_v)

        @pl.loop(0, nchunks)
        def _body(g):
            off = g * chunk
            pltpu.async_copy(
                table_hbm.at[idx_v.at[pl.ds(off, chunk)]], rows, gsem
            ).wait()
            pltpu.sync_copy(rows, out_hbm.at[pl.ds(base + off, chunk)])

    return emb(idx, table)


def kernel(X, table):
    B0, B1 = X.shape
    B = B0 * B1
    idx = X.reshape(B).astype(jnp.int32)
    info = plsc.get_sparse_core_info()
    out = _emb_lookup(
        idx,
        table,
        num_cores=info.num_cores,
        num_subcores=info.num_subcores,
        chunk=512,
    )
    return out.reshape(B0, B1, table.shape[1])


# speedup vs baseline: 1.0259x; 1.0259x over previous
import functools

import jax
import jax.numpy as jnp
from jax import lax
from jax.experimental import pallas as pl
from jax.experimental.pallas import tpu as pltpu
from jax.experimental.pallas import tpu_sc as plsc


def _emb_lookup(idx, table, *, num_cores, num_subcores, chunk):
    B, = idx.shape
    V, D = table.shape
    nw = num_cores * num_subcores
    b_per_w = B // nw
    nchunks = b_per_w // chunk

    mesh = plsc.VectorSubcoreMesh(core_axis_name="c", subcore_axis_name="s")

    @functools.partial(
        pl.kernel,
        out_type=jax.ShapeDtypeStruct((B, D), jnp.float32),
        mesh=mesh,
        compiler_params=pltpu.CompilerParams(use_tc_tiling_on_sc=False),
        scratch_types=[
            pltpu.VMEM((b_per_w,), jnp.int32),
            pltpu.VMEM((chunk, D), jnp.float32),
            pltpu.VMEM((chunk, D), jnp.float32),
            pltpu.SemaphoreType.DMA,
            pltpu.SemaphoreType.DMA,
            pltpu.SemaphoreType.DMA,
            pltpu.SemaphoreType.DMA,
        ],
    )
    def emb(idx_hbm, table_hbm, out_hbm, idx_v, rows0, rows1,
            gsem0, gsem1, ssem0, ssem1):
        wid = lax.axis_index("s") * num_cores + lax.axis_index("c")
        base = wid * b_per_w
        pltpu.sync_copy(idx_hbm.at[pl.ds(base, b_per_w)], idx_v)

        bufs = (rows0, rows1)
        gsems = (gsem0, gsem1)
        ssems = (ssem0, ssem1)

        def start_gather(g, b):
            pltpu.async_copy(
                table_hbm.at[idx_v.at[pl.ds(g * chunk, chunk)]],
                bufs[b], gsems[b])

        def start_store(g, b):
            pltpu.async_copy(
                bufs[b], out_hbm.at[pl.ds(base + g * chunk, chunk)],
                ssems[b])

        def wait_gather(b):
            pltpu.make_async_copy(
                table_hbm.at[pl.ds(0, chunk)], bufs[b], gsems[b]).wait()

        def wait_store(b):
            pltpu.make_async_copy(
                bufs[b], out_hbm.at[pl.ds(base, chunk)], ssems[b]).wait()

        start_gather(0, 0)
        start_gather(1, 1)

        @pl.loop(0, nchunks // 2)
        def _body(o):
            for b in range(2):
                g = o * 2 + b
                wait_gather(b)
                start_store(g, b)

                @pl.when(g + 2 < nchunks)
                def _refill():
                    wait_store(b)
                    start_gather(g + 2, b)

        wait_store(0)
        wait_store(1)

    return emb(idx, table)


def kernel(X, table):
    B0, B1 = X.shape
    B = B0 * B1
    idx = X.reshape(B).astype(jnp.int32)
    info = plsc.get_sparse_core_info()
    out = _emb_lookup(
        idx,
        table,
        num_cores=info.num_cores,
        num_subcores=info.num_subcores,
        chunk=512,
    )
    return out.reshape(B0, B1, table.shape[1])


# speedup vs baseline: 1.0274x; 1.0015x over previous
import functools

import jax
import jax.numpy as jnp
from jax import lax
from jax.experimental import pallas as pl
from jax.experimental.pallas import tpu as pltpu
from jax.experimental.pallas import tpu_sc as plsc


def _emb_lookup(idx, table, *, num_cores, num_subcores, chunk):
    B, = idx.shape
    V, D = table.shape
    nw = num_cores * num_subcores
    b_per_w = B // nw
    nchunks = b_per_w // chunk

    mesh = plsc.VectorSubcoreMesh(core_axis_name="c", subcore_axis_name="s")

    @functools.partial(
        pl.kernel,
        out_type=jax.ShapeDtypeStruct((B, D), jnp.float32),
        mesh=mesh,
        compiler_params=pltpu.CompilerParams(
            use_tc_tiling_on_sc=False, skip_device_barrier=True),
        scratch_types=[
            pltpu.VMEM((b_per_w,), jnp.int32),
            pltpu.VMEM((chunk, D), jnp.float32),
            pltpu.VMEM((chunk, D), jnp.float32),
            pltpu.SemaphoreType.DMA,
            pltpu.SemaphoreType.DMA,
            pltpu.SemaphoreType.DMA,
            pltpu.SemaphoreType.DMA,
        ],
    )
    def emb(idx_hbm, table_hbm, out_hbm, idx_v, rows0, rows1,
            gsem0, gsem1, ssem0, ssem1):
        wid = lax.axis_index("s") * num_cores + lax.axis_index("c")
        base = wid * b_per_w
        pltpu.sync_copy(idx_hbm.at[pl.ds(base, b_per_w)], idx_v)

        bufs = (rows0, rows1)
        gsems = (gsem0, gsem1)
        ssems = (ssem0, ssem1)

        def start_gather(g, b):
            pltpu.async_copy(
                table_hbm.at[idx_v.at[pl.ds(g * chunk, chunk)]],
                bufs[b], gsems[b])

        def start_store(g, b):
            pltpu.async_copy(
                bufs[b], out_hbm.at[pl.ds(base + g * chunk, chunk)],
                ssems[b])

        def wait_gather(b):
            pltpu.make_async_copy(
                table_hbm.at[pl.ds(0, chunk)], bufs[b], gsems[b]).wait()

        def wait_store(b):
            pltpu.make_async_copy(
                bufs[b], out_hbm.at[pl.ds(base, chunk)], ssems[b]).wait()

        start_gather(0, 0)
        start_gather(1, 1)

        @pl.loop(0, nchunks // 2)
        def _body(o):
            for b in range(2):
                g = o * 2 + b
                wait_gather(b)
                start_store(g, b)

                @pl.when(g + 2 < nchunks)
                def _refill():
                    wait_store(b)
                    start_gather(g + 2, b)

        wait_store(0)
        wait_store(1)

    return emb(idx, table)


def kernel(X, table):
    B0, B1 = X.shape
    B = B0 * B1
    idx = X.reshape(B).astype(jnp.int32)
    info = plsc.get_sparse_core_info()
    out = _emb_lookup(
        idx,
        table,
        num_cores=info.num_cores,
        num_subcores=info.num_subcores,
        chunk=512,
    )
    return out.reshape(B0, B1, table.shape[1])


# speedup vs baseline: 1.2506x; 1.2172x over previous
import functools

import jax
import jax.numpy as jnp
from jax import lax
from jax.experimental import pallas as pl
from jax.experimental.pallas import tpu as pltpu
from jax.experimental.pallas import tpu_sc as plsc


def _emb_lookup(idx, table, *, num_cores, num_subcores, chunk, out_dim):
    B, = idx.shape
    V, D = table.shape
    nw = num_cores * num_subcores
    b_per_w = B // nw
    nchunks = b_per_w // chunk

    mesh = plsc.VectorSubcoreMesh(core_axis_name="c", subcore_axis_name="s")

    @functools.partial(
        pl.kernel,
        out_type=jax.ShapeDtypeStruct((B, D), jnp.float32),
        mesh=mesh,
        compiler_params=pltpu.CompilerParams(
            use_tc_tiling_on_sc=True, skip_device_barrier=True),
        scratch_types=[
            pltpu.VMEM((b_per_w,), jnp.int32),
            pltpu.VMEM((chunk, D), jnp.float32),
            pltpu.VMEM((chunk, D), jnp.float32),
            pltpu.SemaphoreType.DMA,
            pltpu.SemaphoreType.DMA,
            pltpu.SemaphoreType.DMA,
            pltpu.SemaphoreType.DMA,
        ],
    )
    def emb(idx_hbm, table_hbm, out_hbm, idx_v, rows0, rows1,
            gsem0, gsem1, ssem0, ssem1):
        wid = lax.axis_index("s") * num_cores + lax.axis_index("c")
        base = wid * b_per_w
        pltpu.sync_copy(idx_hbm.at[pl.ds(base, b_per_w)], idx_v)

        bufs = (rows0, rows1)
        gsems = (gsem0, gsem1)
        ssems = (ssem0, ssem1)

        def start_gather(g, b):
            pltpu.async_copy(
                table_hbm.at[idx_v.at[pl.ds(g * chunk, chunk)]],
                bufs[b], gsems[b])

        def start_store(g, b):
            pltpu.async_copy(
                bufs[b], out_hbm.at[pl.ds(base + g * chunk, chunk)],
                ssems[b])

        def wait_gather(b):
            pltpu.make_async_copy(
                table_hbm.at[pl.ds(0, chunk)], bufs[b], gsems[b]).wait()

        def wait_store(b):
            pltpu.make_async_copy(
                bufs[b], out_hbm.at[pl.ds(base, chunk)], ssems[b]).wait()

        start_gather(0, 0)
        start_gather(1, 1)

        @pl.loop(0, nchunks // 2)
        def _body(o):
            for b in range(2):
                g = o * 2 + b
                wait_gather(b)
                start_store(g, b)

                @pl.when(g + 2 < nchunks)
                def _refill():
                    wait_store(b)
                    start_gather(g + 2, b)

        wait_store(0)
        wait_store(1)

    return emb(idx, table)


def kernel(X, table):
    B0, B1 = X.shape
    B = B0 * B1
    D = table.shape[1]
    idx = X.reshape(B).astype(jnp.int32)
    tbl128 = jnp.pad(table, ((0, 0), (0, 128 - D)))
    info = plsc.get_sparse_core_info()
    out = _emb_lookup(
        idx,
        tbl128,
        num_cores=info.num_cores,
        num_subcores=info.num_subcores,
        chunk=256,
        out_dim=D,
    )
    return out.reshape(B0, B1, 128)[:, :, :D]
